# baseline (speedup 1.0000x reference)
import functools

import jax
import jax.numpy as jnp
from jax import lax
from jax.experimental import pallas as pl
from jax.experimental.pallas import tpu as pltpu
from jax.experimental.pallas import tpu_sc as plsc

N_ROWS = 20000
N_SEL = 1001
NC = 2
NS = 16
NW = NC * NS
B = 1024
BPW = B // NW
INPUT_SIZE = 640.0

S_BASE = 0
B_BASE = N_ROWS
K_BASE = 5 * N_ROWS
SEL_BASE = 15 * N_ROWS
TBL = SEL_BASE + 3000

_mesh = plsc.VectorSubcoreMesh(core_axis_name="c", subcore_axis_name="s")


@functools.partial(
    pl.kernel,
    mesh=_mesh,
    out_type=[
        jax.ShapeDtypeStruct((B,), jnp.float32),
        jax.ShapeDtypeStruct((B * 4,), jnp.int32),
        jax.ShapeDtypeStruct((B * 10,), jnp.int32),
    ],
    scratch_types=[
        pltpu.VMEM((BPW,), jnp.int32),
        pltpu.VMEM((32,), jnp.int32),
        pltpu.VMEM((32,), jnp.int32),
        pltpu.VMEM((BPW,), jnp.float32),
        pltpu.VMEM((BPW,), jnp.int32),
        pltpu.VMEM((BPW,), jnp.float32),
        pltpu.VMEM((128,), jnp.int32),
        pltpu.VMEM((128,), jnp.float32),
        pltpu.VMEM((128,), jnp.int32),
        pltpu.VMEM((128,), jnp.int32),
        pltpu.VMEM((64,), jnp.int32),
        pltpu.VMEM((320,), jnp.float32),
        pltpu.VMEM((128,), jnp.int32),
        pltpu.VMEM((320,), jnp.int32),
        pltpu.SemaphoreType.DMA,
        pltpu.SemaphoreType.DMA,
        pltpu.SemaphoreType.DMA,
        pltpu.SemaphoreType.DMA,
        pltpu.SemaphoreType.DMA,
    ],
)
def _postprocess_sc(org_hbm, tbl_hbm,
                    out_s, out_b, out_k,
                    pos_v, opat_v, org_g, rid_f, idx_v, sg,
                    eb, bg, ek0, ek1, ek2, kg, bout, kout,
                    sem_r, sem_o, sem_s, sem_b, sem_k):
    wid = lax.axis_index("s") * NC + lax.axis_index("c")
    lane = lax.iota(jnp.int32, 16)

    lo = SEL_BASE + 1999
    hi = SEL_BASE + 2999
    for k in range(NW):
        @pl.when(wid == k)
        def _store_pos(k=k):
            pos_v[pl.ds(0, 16)] = jnp.minimum(lane + (lo + 32 * k), hi)
            pos_v[pl.ds(16, 16)] = jnp.minimum(lane + (lo + 32 * k + 16), hi)
    opat_v[pl.ds(0, 16)] = lane * 0 + 1
    opat_v[pl.ds(16, 16)] = lane * 0

    c_o = pltpu.async_copy(org_hbm.at[opat_v], org_g, sem_o)
    c_r = pltpu.async_copy(tbl_hbm.at[pos_v], rid_f, sem_r)
    c_r.wait()

    idx_v[pl.ds(0, 16)] = rid_f[pl.ds(0, 16)].astype(jnp.int32)
    idx_v[pl.ds(16, 16)] = rid_f[pl.ds(16, 16)].astype(jnp.int32)

    @pl.when(wid == 0)
    def _patch():
        idx_v[pl.ds(0, 16)] = jnp.where(lane < 1, 0, idx_v[pl.ds(0, 16)])

    iv0 = idx_v[pl.ds(0, 16)]
    iv1 = idx_v[pl.ds(16, 16)]

    c_s = pltpu.async_copy(tbl_hbm.at[idx_v], sg, sem_s)
    for c in range(4):
        col = B_BASE + (c ^ 1) * N_ROWS
        eb[pl.ds(32 * c, 16)] = iv0 + col
        eb[pl.ds(32 * c + 16, 16)] = iv1 + col
    c_b = pltpu.async_copy(tbl_hbm.at[eb], bg, sem_b)
    for m in range(10):
        col = K_BASE + m * N_ROWS
        if m < 4:
            ek0[pl.ds(32 * m, 16)] = iv0 + col
            ek0[pl.ds(32 * m + 16, 16)] = iv1 + col
        elif m < 8:
            ek1[pl.ds(32 * (m - 4), 16)] = iv0 + col
            ek1[pl.ds(32 * (m - 4) + 16, 16)] = iv1 + col
        else:
            ek2[pl.ds(32 * (m - 8), 16)] = iv0 + col
            ek2[pl.ds(32 * (m - 8) + 16, 16)] = iv1 + col
    c_k0 = pltpu.async_copy(tbl_hbm.at[ek0], kg.at[pl.ds(0, 128)], sem_k)
    c_k1 = pltpu.async_copy(tbl_hbm.at[ek1], kg.at[pl.ds(128, 128)], sem_k)
    c_k2 = pltpu.async_copy(tbl_hbm.at[ek2], kg.at[pl.ds(256, 64)], sem_k)

    c_o.wait()
    w_vec = org_g[pl.ds(0, 16)].astype(jnp.float32)
    h_vec = org_g[pl.ds(16, 16)].astype(jnp.float32)
    m_vec = jnp.maximum(w_vec, h_vec)

    tile = wid >> 2
    rlo = (wid & 3) * 32

    c_b.wait()
    for c in range(4):
        bound = w_vec if c % 2 == 0 else h_vec
        for half in range(2):
            g = bg[pl.ds(32 * c + 16 * half, 16)]
            t = g * m_vec / INPUT_SIZE
            bout[pl.ds(32 * c + 16 * half, 16)] = (
                jnp.clip(t, 0.0, bound).astype(jnp.int32))
        pltpu.sync_copy(
            bout.at[pl.ds(32 * c, 32)],
            out_b.at[pl.ds(tile * 512 + c * 128 + rlo, 32)])

    c_k0.wait()
    c_k1.wait()
    c_k2.wait()
    for m in range(10):
        bound = w_vec if m % 2 == 0 else h_vec
        for half in range(2):
            g = kg[pl.ds(32 * m + 16 * half, 16)]
            t = g * m_vec / INPUT_SIZE
            kout[pl.ds(32 * m + 16 * half, 16)] = (
                jnp.clip(t, 0.0, bound).astype(jnp.int32))
        pltpu.sync_copy(
            kout.at[pl.ds(32 * m, 32)],
            out_k.at[pl.ds((m >> 1) * 2048 + (m & 1) * 128
                           + tile * 256 + rlo, 32)])

    c_s.wait()
    pltpu.sync_copy(sg, out_s.at[pl.ds(wid * BPW, BPW)])


def kernel(org_size, scores, bboxes, kpss, selected_idx):
    tbl = jnp.concatenate([
        scores.reshape(N_ROWS),
        jnp.transpose(bboxes, (0, 2, 1)).reshape(4 * N_ROWS),
        jnp.transpose(kpss, (1, 2, 0)).reshape(10 * N_ROWS),
        jnp.transpose(selected_idx).reshape(3000).astype(jnp.float32),
    ])
    out_s, out_b, out_k = _postprocess_sc(org_size.astype(jnp.int32), tbl)
    bb_o = out_b.reshape(8, 4, 128).transpose(0, 2, 1).reshape(B, 4)
    kp_o = out_k.reshape(5, 8, 2, 128).transpose(1, 3, 0, 2).reshape(B, 5, 2)
    return (out_s[:N_SEL], bb_o[:N_SEL], kp_o[:N_SEL])


# speedup vs baseline: 1.2091x; 1.2091x over previous
import functools

import jax
import jax.numpy as jnp
from jax import lax
from jax.experimental import pallas as pl
from jax.experimental.pallas import tpu as pltpu
from jax.experimental.pallas import tpu_sc as plsc

N_ROWS = 20000
N_SEL = 1001
NC = 2
NS = 16
NW = NC * NS
B = 1024
BPW = B // NW
INPUT_SIZE = 640.0

_mesh = plsc.VectorSubcoreMesh(core_axis_name="c", subcore_axis_name="s")


@functools.partial(
    pl.kernel,
    mesh=_mesh,
    out_type=[
        jax.ShapeDtypeStruct((B,), jnp.float32),
        jax.ShapeDtypeStruct((B * 4,), jnp.int32),
        jax.ShapeDtypeStruct((B * 10,), jnp.int32),
    ],
    scratch_types=[
        pltpu.VMEM((BPW,), jnp.int32),
        pltpu.VMEM((32,), jnp.int32),
        pltpu.VMEM((32,), jnp.int32),
        pltpu.VMEM((BPW,), jnp.int32),
        pltpu.VMEM((BPW,), jnp.float32),
        pltpu.VMEM((128,), jnp.int32),
        pltpu.VMEM((128,), jnp.float32),
        pltpu.VMEM((128,), jnp.int32),
        pltpu.VMEM((128,), jnp.int32),
        pltpu.VMEM((64,), jnp.int32),
        pltpu.VMEM((320,), jnp.float32),
        pltpu.VMEM((128,), jnp.int32),
        pltpu.VMEM((320,), jnp.int32),
        pltpu.SemaphoreType.DMA,
        pltpu.SemaphoreType.DMA,
        pltpu.SemaphoreType.DMA,
        pltpu.SemaphoreType.DMA,
        pltpu.SemaphoreType.DMA,
    ],
)
def _postprocess_sc(org_hbm, sel_hbm, s_hbm, b_hbm, k_hbm,
                    out_s, out_b, out_k,
                    pos_v, opat_v, org_g, idx_v, sg, eb, bg,
                    ek0, ek1, ek2, kg, bout, kout,
                    sem_r, sem_o, sem_s, sem_b, sem_k):
    wid = lax.axis_index("s") * NC + lax.axis_index("c")
    lane = lax.iota(jnp.int32, 16)

    for k in range(NW):
        @pl.when(wid == k)
        def _store_pos(k=k):
            pos_v[pl.ds(0, 16)] = jnp.minimum(lane + (1999 + 32 * k), 2999)
            pos_v[pl.ds(16, 16)] = jnp.minimum(lane + (2015 + 32 * k), 2999)
    opat_v[pl.ds(0, 16)] = lane * 0 + 1
    opat_v[pl.ds(16, 16)] = lane * 0

    c_o = pltpu.async_copy(org_hbm.at[opat_v], org_g, sem_o)
    c_r = pltpu.async_copy(sel_hbm.at[pos_v], idx_v, sem_r)
    c_r.wait()

    @pl.when(wid == 0)
    def _patch():
        idx_v[pl.ds(0, 16)] = jnp.where(lane < 1, 0, idx_v[pl.ds(0, 16)])

    c_s = pltpu.async_copy(s_hbm.at[idx_v], sg, sem_s)
    iv0 = idx_v[pl.ds(0, 16)]
    iv1 = idx_v[pl.ds(16, 16)]
    for c in range(4):
        col = (c ^ 1) * N_ROWS
        eb[pl.ds(32 * c, 16)] = iv0 + col
        eb[pl.ds(32 * c + 16, 16)] = iv1 + col
    c_b = pltpu.async_copy(b_hbm.at[eb], bg, sem_b)
    for m in range(10):
        col = m * N_ROWS
        if m < 4:
            ek0[pl.ds(32 * m, 16)] = iv0 + col
            ek0[pl.ds(32 * m + 16, 16)] = iv1 + col
        elif m < 8:
            ek1[pl.ds(32 * (m - 4), 16)] = iv0 + col
            ek1[pl.ds(32 * (m - 4) + 16, 16)] = iv1 + col
        else:
            ek2[pl.ds(32 * (m - 8), 16)] = iv0 + col
            ek2[pl.ds(32 * (m - 8) + 16, 16)] = iv1 + col
    c_k0 = pltpu.async_copy(k_hbm.at[ek0], kg.at[pl.ds(0, 128)], sem_k)
    c_k1 = pltpu.async_copy(k_hbm.at[ek1], kg.at[pl.ds(128, 128)], sem_k)
    c_k2 = pltpu.async_copy(k_hbm.at[ek2], kg.at[pl.ds(256, 64)], sem_k)

    c_o.wait()
    w_vec = org_g[pl.ds(0, 16)].astype(jnp.float32)
    h_vec = org_g[pl.ds(16, 16)].astype(jnp.float32)
    m_vec = jnp.maximum(w_vec, h_vec)

    tile = wid >> 2
    rlo = (wid & 3) * 32

    c_b.wait()
    for c in range(4):
        bound = w_vec if c % 2 == 0 else h_vec
        for half in range(2):
            g = bg[pl.ds(32 * c + 16 * half, 16)]
            t = g * m_vec / INPUT_SIZE
            bout[pl.ds(32 * c + 16 * half, 16)] = (
                jnp.clip(t, 0.0, bound).astype(jnp.int32))
        pltpu.sync_copy(
            bout.at[pl.ds(32 * c, 32)],
            out_b.at[pl.ds(tile * 512 + c * 128 + rlo, 32)])

    c_k0.wait()
    c_k1.wait()
    c_k2.wait()
    for m in range(10):
        bound = w_vec if m % 2 == 0 else h_vec
        for half in range(2):
            g = kg[pl.ds(32 * m + 16 * half, 16)]
            t = g * m_vec / INPUT_SIZE
            kout[pl.ds(32 * m + 16 * half, 16)] = (
                jnp.clip(t, 0.0, bound).astype(jnp.int32))
        pltpu.sync_copy(
            kout.at[pl.ds(32 * m, 32)],
            out_k.at[pl.ds((m >> 1) * 2048 + (m & 1) * 128
                           + tile * 256 + rlo, 32)])

    c_s.wait()
    pltpu.sync_copy(sg, out_s.at[pl.ds(wid * BPW, BPW)])


def kernel(org_size, scores, bboxes, kpss, selected_idx):
    sel_t = jnp.transpose(selected_idx.astype(jnp.int32)).reshape(3000)
    bb_t = jnp.transpose(bboxes, (0, 2, 1)).reshape(4 * N_ROWS)
    kp_t = jnp.transpose(kpss, (1, 2, 0)).reshape(10 * N_ROWS)
    out_s, out_b, out_k = _postprocess_sc(
        org_size.astype(jnp.int32),
        sel_t,
        scores.reshape(N_ROWS),
        bb_t,
        kp_t,
    )
    bb_o = out_b.reshape(8, 4, 128).transpose(0, 2, 1).reshape(B, 4)
    kp_o = out_k.reshape(5, 8, 2, 128).transpose(1, 3, 0, 2).reshape(B, 5, 2)
    return (out_s[:N_SEL], bb_o[:N_SEL], kp_o[:N_SEL])


# speedup vs baseline: 1.2393x; 1.0250x over previous
import functools

import jax
import jax.numpy as jnp
from jax import lax
from jax.experimental import pallas as pl
from jax.experimental.pallas import tpu as pltpu
from jax.experimental.pallas import tpu_sc as plsc

N_ROWS = 20000
N_SEL = 1001
NC = 2
NS = 16
NW = NC * NS
B = 1024
BPW = B // NW
INPUT_SIZE = 640.0

_mesh = plsc.VectorSubcoreMesh(core_axis_name="c", subcore_axis_name="s")


@functools.partial(
    pl.kernel,
    mesh=_mesh,
    out_type=[
        jax.ShapeDtypeStruct((B,), jnp.float32),
        jax.ShapeDtypeStruct((B * 4,), jnp.int32),
        jax.ShapeDtypeStruct((B * 10,), jnp.int32),
    ],
    scratch_types=[
        pltpu.VMEM((BPW,), jnp.int32),
        pltpu.VMEM((32,), jnp.int32),
        pltpu.VMEM((32,), jnp.int32),
        pltpu.VMEM((BPW,), jnp.int32),
        pltpu.VMEM((BPW,), jnp.float32),
        pltpu.VMEM((128,), jnp.int32),
        pltpu.VMEM((128,), jnp.float32),
        pltpu.VMEM((128,), jnp.int32),
        pltpu.VMEM((128,), jnp.int32),
        pltpu.VMEM((64,), jnp.int32),
        pltpu.VMEM((320,), jnp.float32),
        pltpu.VMEM((128,), jnp.int32),
        pltpu.VMEM((320,), jnp.int32),
        pltpu.SemaphoreType.DMA,
        pltpu.SemaphoreType.DMA,
        pltpu.SemaphoreType.DMA,
        pltpu.SemaphoreType.DMA,
        pltpu.SemaphoreType.DMA,
        pltpu.SemaphoreType.DMA,
    ],
)
def _postprocess_sc(org_hbm, sel_hbm, s_hbm, b_hbm, k_hbm,
                    out_s, out_b, out_k,
                    pos_v, opat_v, org_g, idx_v, sg, eb, bg,
                    ek0, ek1, ek2, kg, bout, kout,
                    sem_r, sem_o, sem_s, sem_b, sem_k, sem_w):
    wid = lax.axis_index("s") * NC + lax.axis_index("c")
    lane = lax.iota(jnp.int32, 16)

    for k in range(NW):
        @pl.when(wid == k)
        def _store_pos(k=k):
            pos_v[pl.ds(0, 16)] = jnp.minimum(lane + (1999 + 32 * k), 2999)
            pos_v[pl.ds(16, 16)] = jnp.minimum(lane + (2015 + 32 * k), 2999)
    opat_v[pl.ds(0, 16)] = lane * 0 + 1
    opat_v[pl.ds(16, 16)] = lane * 0

    c_o = pltpu.async_copy(org_hbm.at[opat_v], org_g, sem_o)
    c_r = pltpu.async_copy(sel_hbm.at[pos_v], idx_v, sem_r)
    c_r.wait()

    @pl.when(wid == 0)
    def _patch():
        idx_v[pl.ds(0, 16)] = jnp.where(lane < 1, 0, idx_v[pl.ds(0, 16)])

    c_s = pltpu.async_copy(s_hbm.at[idx_v], sg, sem_s)
    iv0 = idx_v[pl.ds(0, 16)]
    iv1 = idx_v[pl.ds(16, 16)]
    for c in range(4):
        col = (c ^ 1) * N_ROWS
        eb[pl.ds(32 * c, 16)] = iv0 + col
        eb[pl.ds(32 * c + 16, 16)] = iv1 + col
    c_b = pltpu.async_copy(b_hbm.at[eb], bg, sem_b)
    for m in range(10):
        col = m * N_ROWS
        if m < 4:
            ek0[pl.ds(32 * m, 16)] = iv0 + col
            ek0[pl.ds(32 * m + 16, 16)] = iv1 + col
        elif m < 8:
            ek1[pl.ds(32 * (m - 4), 16)] = iv0 + col
            ek1[pl.ds(32 * (m - 4) + 16, 16)] = iv1 + col
        else:
            ek2[pl.ds(32 * (m - 8), 16)] = iv0 + col
            ek2[pl.ds(32 * (m - 8) + 16, 16)] = iv1 + col
    c_k0 = pltpu.async_copy(k_hbm.at[ek0], kg.at[pl.ds(0, 128)], sem_k)
    c_k1 = pltpu.async_copy(k_hbm.at[ek1], kg.at[pl.ds(128, 128)], sem_k)
    c_k2 = pltpu.async_copy(k_hbm.at[ek2], kg.at[pl.ds(256, 64)], sem_k)

    c_o.wait()
    w_vec = org_g[pl.ds(0, 16)].astype(jnp.float32)
    h_vec = org_g[pl.ds(16, 16)].astype(jnp.float32)
    m_vec = jnp.maximum(w_vec, h_vec)

    tile = wid >> 2
    rlo = (wid & 3) * 32

    outs = []
    c_b.wait()
    for c in range(4):
        bound = w_vec if c % 2 == 0 else h_vec
        for half in range(2):
            g = bg[pl.ds(32 * c + 16 * half, 16)]
            t = g * m_vec / INPUT_SIZE
            bout[pl.ds(32 * c + 16 * half, 16)] = (
                jnp.clip(t, 0.0, bound).astype(jnp.int32))
        outs.append(pltpu.async_copy(
            bout.at[pl.ds(32 * c, 32)],
            out_b.at[pl.ds(tile * 512 + c * 128 + rlo, 32)], sem_w))

    c_k0.wait()
    c_k1.wait()
    c_k2.wait()
    for m in range(10):
        bound = w_vec if m % 2 == 0 else h_vec
        for half in range(2):
            g = kg[pl.ds(32 * m + 16 * half, 16)]
            t = g * m_vec / INPUT_SIZE
            kout[pl.ds(32 * m + 16 * half, 16)] = (
                jnp.clip(t, 0.0, bound).astype(jnp.int32))
        outs.append(pltpu.async_copy(
            kout.at[pl.ds(32 * m, 32)],
            out_k.at[pl.ds((m >> 1) * 2048 + (m & 1) * 128
                           + tile * 256 + rlo, 32)], sem_w))

    c_s.wait()
    outs.append(pltpu.async_copy(sg, out_s.at[pl.ds(wid * BPW, BPW)], sem_w))
    for c_ in outs:
        c_.wait()


def kernel(org_size, scores, bboxes, kpss, selected_idx):
    sel_t = jnp.transpose(selected_idx.astype(jnp.int32)).reshape(3000)
    bb_t = jnp.transpose(bboxes, (0, 2, 1)).reshape(4 * N_ROWS)
    kp_t = jnp.transpose(kpss, (1, 2, 0)).reshape(10 * N_ROWS)
    out_s, out_b, out_k = _postprocess_sc(
        org_size.astype(jnp.int32),
        sel_t,
        scores.reshape(N_ROWS),
        bb_t,
        kp_t,
    )
    bb_o = out_b.reshape(8, 4, 128).transpose(0, 2, 1).reshape(B, 4)
    kp_o = out_k.reshape(5, 8, 2, 128).transpose(1, 3, 0, 2).reshape(B, 5, 2)
    return (out_s[:N_SEL], bb_o[:N_SEL], kp_o[:N_SEL])


# speedup vs baseline: 1.4086x; 1.1366x over previous
import functools

import jax
import jax.numpy as jnp
from jax import lax
from jax.experimental import pallas as pl
from jax.experimental.pallas import tpu as pltpu
from jax.experimental.pallas import tpu_sc as plsc

N_ROWS = 20000
N_SEL = 1001
NC = 1
NS = 16
NW = NC * NS
B = 1024
BPW = B // NW
NQ = BPW // 16
INPUT_SIZE = 640.0

_mesh = plsc.VectorSubcoreMesh(
    core_axis_name="c", subcore_axis_name="s", num_cores=NC)


def _chunk_sizes(total):
    sizes = []
    left = total
    while left > 0:
        sizes.append(min(128, left))
        left -= 128
    return sizes


_BCH = _chunk_sizes(4 * BPW)
_KCH = _chunk_sizes(10 * BPW)


@functools.partial(
    pl.kernel,
    mesh=_mesh,
    out_type=[
        jax.ShapeDtypeStruct((B,), jnp.float32),
        jax.ShapeDtypeStruct((B * 4,), jnp.int32),
        jax.ShapeDtypeStruct((B * 10,), jnp.int32),
    ],
    scratch_types=(
        [
            pltpu.VMEM((BPW,), jnp.int32),
            pltpu.VMEM((32,), jnp.int32),
            pltpu.VMEM((32,), jnp.int32),
            pltpu.VMEM((BPW,), jnp.int32),
            pltpu.VMEM((BPW,), jnp.float32),
        ]
        + [pltpu.VMEM((n,), jnp.int32) for n in _BCH]
        + [pltpu.VMEM((n,), jnp.int32) for n in _KCH]
        + [
            pltpu.VMEM((4 * BPW,), jnp.float32),
            pltpu.VMEM((10 * BPW,), jnp.float32),
            pltpu.VMEM((4 * BPW,), jnp.int32),
            pltpu.VMEM((10 * BPW,), jnp.int32),
            pltpu.SemaphoreType.DMA,
            pltpu.SemaphoreType.DMA,
            pltpu.SemaphoreType.DMA,
            pltpu.SemaphoreType.DMA,
            pltpu.SemaphoreType.DMA,
            pltpu.SemaphoreType.DMA,
        ]
    ),
)
def _postprocess_sc(org_hbm, sel_hbm, s_hbm, b_hbm, k_hbm,
                    out_s, out_b, out_k,
                    pos_v, opat_v, org_g, idx_v, sg, *rest):
    ebs = rest[:len(_BCH)]
    eks = rest[len(_BCH):len(_BCH) + len(_KCH)]
    (bg, kg, bout, kout,
     sem_r, sem_o, sem_s, sem_b, sem_k, sem_w) = rest[len(_BCH) + len(_KCH):]

    wid = lax.axis_index("s") * NC + lax.axis_index("c")
    lane = lax.iota(jnp.int32, 16)

    for k in range(NW):
        @pl.when(wid == k)
        def _store_pos(k=k):
            for q in range(NQ):
                pos_v[pl.ds(16 * q, 16)] = jnp.minimum(
                    lane + (1999 + BPW * k + 16 * q), 2999)
    opat_v[pl.ds(0, 16)] = lane * 0 + 1
    opat_v[pl.ds(16, 16)] = lane * 0

    c_o = pltpu.async_copy(org_hbm.at[opat_v], org_g, sem_o)
    c_r = pltpu.async_copy(sel_hbm.at[pos_v], idx_v, sem_r)
    c_r.wait()

    @pl.when(wid == 0)
    def _patch():
        idx_v[pl.ds(0, 16)] = jnp.where(lane < 1, 0, idx_v[pl.ds(0, 16)])

    c_s = pltpu.async_copy(s_hbm.at[idx_v], sg, sem_s)
    ivs = [idx_v[pl.ds(16 * q, 16)] for q in range(NQ)]

    def store_col_indices(refs, ncols, col_of):
        for c in range(ncols):
            for q in range(NQ):
                flat = c * BPW + 16 * q
                refs[flat // 128][pl.ds(flat % 128, 16)] = (
                    ivs[q] + col_of(c) * N_ROWS)

    store_col_indices(ebs, 4, lambda c: c ^ 1)
    c_bs = [
        pltpu.async_copy(
            b_hbm.at[ebs[i]], bg.at[pl.ds(128 * i, _BCH[i])], sem_b)
        for i in range(len(_BCH))
    ]
    store_col_indices(eks, 10, lambda c: c)
    c_ks = [
        pltpu.async_copy(
            k_hbm.at[eks[i]], kg.at[pl.ds(128 * i, _KCH[i])], sem_k)
        for i in range(len(_KCH))
    ]

    c_o.wait()
    w_vec = org_g[pl.ds(0, 16)].astype(jnp.float32)
    h_vec = org_g[pl.ds(16, 16)].astype(jnp.float32)
    m_vec = jnp.maximum(w_vec, h_vec)

    r0 = wid * BPW
    tile = wid // (128 // BPW)
    rlo = (wid % (128 // BPW)) * BPW
    outs = []

    for c_ in c_bs:
        c_.wait()
    for c in range(4):
        bound = w_vec if c % 2 == 0 else h_vec
        for q in range(NQ):
            g = bg[pl.ds(c * BPW + 16 * q, 16)]
            t = g * m_vec / INPUT_SIZE
            bout[pl.ds(c * BPW + 16 * q, 16)] = (
                jnp.clip(t, 0.0, bound).astype(jnp.int32))
        outs.append(pltpu.async_copy(
            bout.at[pl.ds(c * BPW, BPW)],
            out_b.at[pl.ds(tile * 512 + c * 128 + rlo, BPW)],
            sem_w))

    for c_ in c_ks:
        c_.wait()
    for m in range(10):
        bound = w_vec if m % 2 == 0 else h_vec
        for q in range(NQ):
            g = kg[pl.ds(m * BPW + 16 * q, 16)]
            t = g * m_vec / INPUT_SIZE
            kout[pl.ds(m * BPW + 16 * q, 16)] = (
                jnp.clip(t, 0.0, bound).astype(jnp.int32))
        outs.append(pltpu.async_copy(
            kout.at[pl.ds(m * BPW, BPW)],
            out_k.at[pl.ds((m >> 1) * 2048 + (m & 1) * 128
                           + tile * 256 + rlo, BPW)],
            sem_w))

    c_s.wait()
    outs.append(pltpu.async_copy(sg, out_s.at[pl.ds(r0, BPW)], sem_w))
    for c_ in outs:
        c_.wait()


def kernel(org_size, scores, bboxes, kpss, selected_idx):
    sel_t = jnp.transpose(selected_idx.astype(jnp.int32)).reshape(3000)
    bb_t = jnp.transpose(bboxes, (0, 2, 1)).reshape(4 * N_ROWS)
    kp_t = jnp.transpose(kpss, (1, 2, 0)).reshape(10 * N_ROWS)
    out_s, out_b, out_k = _postprocess_sc(
        org_size.astype(jnp.int32),
        sel_t,
        scores.reshape(N_ROWS),
        bb_t,
        kp_t,
    )
    bb_o = out_b.reshape(8, 4, 128).transpose(0, 2, 1).reshape(B, 4)
    kp_o = out_k.reshape(5, 8, 2, 128).transpose(1, 3, 0, 2).reshape(B, 5, 2)
    return (out_s[:N_SEL], bb_o[:N_SEL], kp_o[:N_SEL])
